# baseline (speedup 1.0000x reference)
import jax
import jax.numpy as jnp
from jax import lax
from jax.experimental import pallas as pl
from jax.experimental.pallas import tpu as pltpu
from jax.experimental.pallas import tpu_sc as plsc

N_NODES = 10000
N_EDGES = 320000
D = 128
DH = D // 2

NC = 2
NS = 16
CHUNK = 128
EPT = N_EDGES // NS
NCHUNKS = 158
EPT_PAD = NCHUNKS * CHUNK
E_PAD = NS * EPT_PAD
ROWS_PER_TILE = 640
N_PAD = NS * ROWS_PER_TILE
ZROWS = 64


def _sc_body(h2_ref, src2_ref, dst_ref, w_ref, out_ref,
             src_v, dst_v, w_v,
             rows_v, rows2_v, rows3_v, rows4_v,
             acc,
             gsem, gsem2, gsem3, gsem4,
             ssem, ssem2, ssem3, ssem4):
    c = lax.axis_index("c")
    s = lax.axis_index("s")

    pltpu.sync_copy(src2_ref.at[s], src_v)
    pltpu.sync_copy(dst_ref.at[s], dst_v)
    pltpu.sync_copy(w_ref.at[s], w_v)

    @plsc.parallel_loop(0, NCHUNKS * (CHUNK // 16), unroll=4)
    def idx_body(g):
        j = g // (CHUNK // 16)
        sl = pl.ds((g % (CHUNK // 16)) * 16, 16)
        src_v[j, sl] = src_v[j, sl] + c

    zf = jnp.zeros((16,), jnp.float32)

    def zrow(r, carry):
        for g in range(DH // 16):
            rows_v[r, pl.ds(g * 16, 16)] = zf
        return carry

    lax.fori_loop(0, CHUNK, zrow, 0)
    for k in range(ROWS_PER_TILE // CHUNK):
        pltpu.sync_copy(rows_v, acc.at[pl.ds(s * ROWS_PER_TILE + k * CHUNK, CHUNK)])
    plsc.subcore_barrier()

    bufs = (rows_v, rows2_v, rows3_v, rows4_v)
    gsems = (gsem, gsem2, gsem3, gsem4)
    ssems = (ssem, ssem2, ssem3, ssem4)
    NB = 4

    def start_gather(j, b):
        pltpu.async_copy(h2_ref.at[src_v.at[j]], bufs[b], gsems[b])

    def wait_gather(b):
        pltpu.make_async_copy(h2_ref.at[src_v.at[0]], bufs[b], gsems[b]).wait()

    def start_scatter(j, b):
        pltpu.async_copy(bufs[b], acc.at[dst_v.at[j]], ssems[b], add=True)

    def wait_scatter(b):
        pltpu.make_async_copy(bufs[b], acc.at[dst_v.at[0]], ssems[b]).wait()

    def scale(j, b):
        buf = bufs[b]

        @plsc.parallel_loop(0, CHUNK // 32, unroll=2)
        def group_body(g):
            e0 = g * 32
            wvec32 = w_v[j, pl.ds(e0, 32)]
            wev, wod = plsc.unpack(wvec32, format=plsc.PackFormat.INTERLEAVED)
            for i in range(32):
                half = wev if i % 2 == 0 else wod
                ii = jnp.full((16,), i // 2, jnp.int32)
                wspl = half.at[ii].get(mode="promise_in_bounds")
                for f in range(DH // 16):
                    sl = pl.ds(f * 16, 16)
                    buf[e0 + i, sl] = buf[e0 + i, sl] * wspl

    for j in range(2):
        start_gather(j, j)

    for j in range(2):
        wait_gather(j)
        scale(j, j)
        start_scatter(j, j)
        start_gather(j + 2, j + 2)

    def quad_body(p, carry):
        for ph in range(NB):
            j = NB * p + 2 + ph
            b = (2 + ph) % NB
            bn = (b + 2) % NB
            wait_gather(b)
            scale(j, b)
            start_scatter(j, b)
            wait_scatter(bn)
            start_gather(jnp.minimum(j + 2, NCHUNKS - 1), bn)
        return carry

    lax.fori_loop(0, (NCHUNKS - 2) // NB, quad_body, 0)

    for b in (2, 3):
        wait_gather(b)
    wait_scatter(0)
    wait_scatter(1)
    plsc.subcore_barrier()

    row0 = s * ROWS_PER_TILE
    pltpu.sync_copy(acc.at[pl.ds(row0, ROWS_PER_TILE)],
                    out_ref.at[c, pl.ds(row0, ROWS_PER_TILE)])


_sc_scatter = pl.kernel(
    _sc_body,
    out_type=jax.ShapeDtypeStruct((NC, N_PAD, DH), jnp.float32),
    mesh=plsc.VectorSubcoreMesh(core_axis_name="c", subcore_axis_name="s"),
    compiler_params=pltpu.CompilerParams(use_tc_tiling_on_sc=False, needs_layout_passes=False),
    scratch_types=[
        pltpu.VMEM((NCHUNKS, CHUNK), jnp.int32),
        pltpu.VMEM((NCHUNKS, CHUNK), jnp.int32),
        pltpu.VMEM((NCHUNKS, CHUNK), jnp.bfloat16),
        pltpu.VMEM((CHUNK, DH), jnp.float32),
        pltpu.VMEM((CHUNK, DH), jnp.float32),
        pltpu.VMEM((CHUNK, DH), jnp.float32),
        pltpu.VMEM((CHUNK, DH), jnp.float32),
        pltpu.VMEM_SHARED((N_PAD, DH), jnp.float32),
        pltpu.SemaphoreType.DMA,
        pltpu.SemaphoreType.DMA,
        pltpu.SemaphoreType.DMA,
        pltpu.SemaphoreType.DMA,
        pltpu.SemaphoreType.DMA,
        pltpu.SemaphoreType.DMA,
        pltpu.SemaphoreType.DMA,
        pltpu.SemaphoreType.DMA,
    ],
)


def _tc_body(p_ref, wt_ref, b_ref, o_ref):
    hcat = jnp.concatenate([p_ref[0], p_ref[1]], axis=-1)
    o_ref[...] = (
        jnp.dot(hcat, wt_ref[...], preferred_element_type=jnp.float32)
        + b_ref[...]
    )


BN = 400


def _linear(partials, Wt, b2):
    return pl.pallas_call(
        _tc_body,
        out_shape=jax.ShapeDtypeStruct((N_NODES, D), jnp.float32),
        grid=(N_NODES // BN,),
        in_specs=[
            pl.BlockSpec((NC, BN, DH), lambda i: (0, i, 0)),
            pl.BlockSpec((D, D), lambda i: (0, 0)),
            pl.BlockSpec((1, D), lambda i: (0, 0)),
        ],
        out_specs=pl.BlockSpec((BN, D), lambda i: (i, 0)),
    )(partials, Wt, b2)


def kernel(h, edge_index, eweight, W, b):
    src = edge_index[0].astype(jnp.int32)
    dst = edge_index[1].astype(jnp.int32)
    w = eweight[:, 0].astype(jnp.bfloat16)

    pad = E_PAD - N_EDGES
    src2 = jnp.concatenate([src * 2, jnp.zeros((pad,), jnp.int32)])
    dst = jnp.concatenate([dst, jnp.zeros((pad,), jnp.int32)])
    w = jnp.concatenate([w, jnp.zeros((pad,), jnp.bfloat16)])

    src2_r = src2.reshape(NS, NCHUNKS, CHUNK)
    dst_r = dst.reshape(NS, NCHUNKS, CHUNK)
    w_r = w.reshape(NS, NCHUNKS, CHUNK)
    h2 = h.reshape(2 * N_NODES, DH)

    partials = _sc_scatter(h2, src2_r, dst_r, w_r)
    return _linear(partials, W.T, b.reshape(1, D))


# speedup vs baseline: 1.3017x; 1.3017x over previous
import jax
import jax.numpy as jnp
from jax import lax
from jax.experimental import pallas as pl
from jax.experimental.pallas import tpu as pltpu
from jax.experimental.pallas import tpu_sc as plsc

N_NODES = 10000
N_EDGES = 320000
D = 128
DH = D // 2

NC = 2
NS = 16
CHUNK = 128
EPT = N_EDGES // NS
NCHUNKS = -(-EPT // CHUNK)
EPT_PAD = NCHUNKS * CHUNK
E_PAD = NS * EPT_PAD
ROWS_PER_TILE = 640
N_PAD = NS * ROWS_PER_TILE
ZROWS = 64


def _sc_body(h2_ref, src2_ref, dst_ref, w_ref, out_ref,
             src_v, dst_v, w_v, rows_v, rows2_v, rows3_v, zbuf, acc,
             gsem, gsem2, gsem3, ssem, ssem2, ssem3):
    c = lax.axis_index("c")
    s = lax.axis_index("s")

    pltpu.sync_copy(src2_ref.at[s], src_v)
    pltpu.sync_copy(dst_ref.at[s], dst_v)
    pltpu.sync_copy(w_ref.at[s], w_v)

    @plsc.parallel_loop(0, NCHUNKS * (CHUNK // 16), unroll=4)
    def idx_body(g):
        j = g // (CHUNK // 16)
        sl = pl.ds((g % (CHUNK // 16)) * 16, 16)
        src_v[j, sl] = src_v[j, sl] + c

    zf = jnp.zeros((16,), jnp.float32)

    def zrow(r, carry):
        for g in range(DH // 16):
            zbuf[r, pl.ds(g * 16, 16)] = zf
        return carry

    lax.fori_loop(0, ZROWS, zrow, 0)
    for k in range(ROWS_PER_TILE // ZROWS):
        pltpu.sync_copy(zbuf, acc.at[pl.ds(s * ROWS_PER_TILE + k * ZROWS, ZROWS)])
    plsc.subcore_barrier()

    bufs = (rows_v, rows2_v, rows3_v)
    gsems = (gsem, gsem2, gsem3)
    ssems = (ssem, ssem2, ssem3)

    def start_gather(j, b):
        pltpu.async_copy(h2_ref.at[src_v.at[j]], bufs[b], gsems[b])

    def wait_gather(b):
        pltpu.make_async_copy(h2_ref.at[src_v.at[0]], bufs[b], gsems[b]).wait()

    def start_scatter(j, b):
        pltpu.async_copy(bufs[b], acc.at[dst_v.at[j]], ssems[b], add=True)

    def wait_scatter(b):
        pltpu.make_async_copy(bufs[b], acc.at[dst_v.at[0]], ssems[b]).wait()

    def scale(j, b):
        buf = bufs[b]

        @plsc.parallel_loop(0, CHUNK // 16, unroll=2)
        def group_body(g):
            e0 = g * 16
            wvec = w_v[j, pl.ds(e0, 16)]
            for i in range(16):
                ii = jnp.full((16,), i, jnp.int32)
                wspl = wvec.at[ii].get(mode="promise_in_bounds")
                for f in range(DH // 16):
                    sl = pl.ds(f * 16, 16)
                    buf[e0 + i, sl] = buf[e0 + i, sl] * wspl

    start_gather(0, 0)
    start_gather(1, 1)

    wait_gather(0)
    scale(0, 0)
    start_scatter(0, 0)
    start_gather(2, 2)

    def trip_body(p, carry):
        for ph in range(3):
            j = 3 * p + 1 + ph
            b = (1 + ph) % 3
            bn = (b + 2) % 3
            wait_gather(b)
            scale(j, b)
            start_scatter(j, b)
            wait_scatter(bn)
            start_gather(jnp.minimum(j + 2, NCHUNKS - 1), bn)
        return carry

    lax.fori_loop(0, (NCHUNKS - 1) // 3, trip_body, 0)

    wait_gather(1)
    wait_gather(2)
    wait_scatter(0)
    plsc.subcore_barrier()

    row0 = s * ROWS_PER_TILE
    pltpu.sync_copy(acc.at[pl.ds(row0, ROWS_PER_TILE)],
                    out_ref.at[c, pl.ds(row0, ROWS_PER_TILE)])


_sc_scatter = pl.kernel(
    _sc_body,
    out_type=jax.ShapeDtypeStruct((NC, N_PAD, DH), jnp.float32),
    mesh=plsc.VectorSubcoreMesh(core_axis_name="c", subcore_axis_name="s"),
    compiler_params=pltpu.CompilerParams(use_tc_tiling_on_sc=False),
    scratch_types=[
        pltpu.VMEM((NCHUNKS, CHUNK), jnp.int32),
        pltpu.VMEM((NCHUNKS, CHUNK), jnp.int32),
        pltpu.VMEM((NCHUNKS, CHUNK), jnp.float32),
        pltpu.VMEM((CHUNK, DH), jnp.float32),
        pltpu.VMEM((CHUNK, DH), jnp.float32),
        pltpu.VMEM((CHUNK, DH), jnp.float32),
        pltpu.VMEM((ZROWS, DH), jnp.float32),
        pltpu.VMEM_SHARED((N_PAD, DH), jnp.float32),
        pltpu.SemaphoreType.DMA,
        pltpu.SemaphoreType.DMA,
        pltpu.SemaphoreType.DMA,
        pltpu.SemaphoreType.DMA,
        pltpu.SemaphoreType.DMA,
        pltpu.SemaphoreType.DMA,
    ],
)


def _tc_body(p_ref, wt_ref, b_ref, o_ref):
    hcat = jnp.concatenate([p_ref[0], p_ref[1]], axis=-1)
    o_ref[...] = (
        jnp.dot(hcat, wt_ref[...], preferred_element_type=jnp.float32)
        + b_ref[...]
    )


BN = 400


def _linear(partials, Wt, b2):
    return pl.pallas_call(
        _tc_body,
        out_shape=jax.ShapeDtypeStruct((N_NODES, D), jnp.float32),
        grid=(N_NODES // BN,),
        in_specs=[
            pl.BlockSpec((NC, BN, DH), lambda i: (0, i, 0)),
            pl.BlockSpec((D, D), lambda i: (0, 0)),
            pl.BlockSpec((1, D), lambda i: (0, 0)),
        ],
        out_specs=pl.BlockSpec((BN, D), lambda i: (i, 0)),
    )(partials, Wt, b2)


def kernel(h, edge_index, eweight, W, b):
    src = edge_index[0].astype(jnp.int32)
    dst = edge_index[1].astype(jnp.int32)
    w = eweight[:, 0].astype(jnp.float32)

    pad = E_PAD - N_EDGES
    src2 = jnp.concatenate([src * 2, jnp.zeros((pad,), jnp.int32)])
    dst = jnp.concatenate([dst, jnp.zeros((pad,), jnp.int32)])
    w = jnp.concatenate([w, jnp.zeros((pad,), jnp.float32)])

    src2_r = src2.reshape(NS, NCHUNKS, CHUNK)
    dst_r = dst.reshape(NS, NCHUNKS, CHUNK)
    w_r = w.reshape(NS, NCHUNKS, CHUNK)
    h2 = h.reshape(2 * N_NODES, DH)

    partials = _sc_scatter(h2, src2_r, dst_r, w_r)
    return _linear(partials, W.T, b.reshape(1, D))
